# Initial kernel scaffold; baseline (speedup 1.0000x reference)
import jax
import jax.numpy as jnp
from jax.experimental import pallas as pl


def kernel(x, edge_index, W1l, W1r, b1, W2l, W2r, b2, Wc, bc):
    raise NotImplementedError("write your pallas kernel here")


# speedup vs baseline: 2.7503x; 2.7503x over previous
import functools

import jax
import jax.numpy as jnp
from jax import lax
from jax.experimental import pallas as pl
from jax.experimental.pallas import tpu as pltpu
from jax.experimental.pallas import tpu_sc as plsc

NC = 2
NS = 16
L = 16
NW = NC * NS


@functools.partial(jax.jit, static_argnums=(3, 4, 5))
def _sc_segsum(src_flat, dst_flat, table, n_pad, rows_per_tile, with_counts):
    d = table.shape[1]
    bsz = 128
    n_steps = rows_per_tile * 128 // bsz
    zrows = n_pad // NS
    nz = zrows // bsz
    zrem = zrows % bsz

    mesh = plsc.VectorSubcoreMesh(
        core_axis_name="c", subcore_axis_name="s", num_cores=NC, num_subcores=NS)

    out_type = [jax.ShapeDtypeStruct((NC, n_pad, d), jnp.float32)]
    scratch = [
        pltpu.VMEM((bsz,), jnp.int32),
        pltpu.VMEM((bsz,), jnp.int32),
        pltpu.VMEM((bsz, d), jnp.float32),
        pltpu.VMEM_SHARED((n_pad, d), jnp.float32),
        pltpu.SemaphoreType.DMA,
    ]
    if with_counts:
        out_type.append(jax.ShapeDtypeStruct((NC, n_pad, d), jnp.float32))

    def body(src_hbm, dst_hbm, tab_hbm, agg_out, *rest):
        if with_counts:
            cnt_out, src_v, dst_v, rows_v, agg_sh, sem = rest
        else:
            src_v, dst_v, rows_v, agg_sh, sem = rest
        cid = lax.axis_index("c")
        sid = lax.axis_index("s")
        wid = sid * NC + cid

        def zr(i, carry):
            for k in range(d // L):
                rows_v[i, pl.ds(k * L, L)] = jnp.zeros((L,), jnp.float32)
            return carry
        lax.fori_loop(0, bsz, zr, 0)

        base = sid * zrows
        def zs(k, carry):
            pltpu.sync_copy(rows_v, agg_sh.at[pl.ds(base + k * bsz, bsz)])
            return carry
        lax.fori_loop(0, nz, zs, 0)
        if zrem:
            pltpu.sync_copy(rows_v.at[pl.ds(0, zrem)],
                            agg_sh.at[pl.ds(base + nz * bsz, zrem)])

        plsc.subcore_barrier()

        def step(j, carry):
            off = pl.multiple_of(wid * rows_per_tile * 128 + j * bsz, bsz)
            pltpu.sync_copy(src_hbm.at[pl.ds(off, bsz)], src_v)
            pltpu.sync_copy(dst_hbm.at[pl.ds(off, bsz)], dst_v)
            pltpu.async_copy(tab_hbm.at[src_v], rows_v, sem).wait()
            pltpu.sync_copy(rows_v, agg_sh.at[dst_v], add=True)
            return carry
        lax.fori_loop(0, n_steps, step, 0)

        plsc.subcore_barrier()

        pltpu.sync_copy(agg_sh.at[pl.ds(base, zrows)],
                        agg_out.at[cid, pl.ds(base, zrows)])

        if with_counts:
            lax.fori_loop(0, bsz, zr, 0)
            lax.fori_loop(0, nz, zs, 0)
            if zrem:
                pltpu.sync_copy(rows_v.at[pl.ds(0, zrem)],
                                agg_sh.at[pl.ds(base + nz * bsz, zrem)])

            def zo(i, carry):
                for k in range(d // L):
                    rows_v[i, pl.ds(k * L, L)] = jnp.ones((L,), jnp.float32)
                return carry
            lax.fori_loop(0, bsz, zo, 0)
            plsc.subcore_barrier()

            def step2(j, carry):
                off = pl.multiple_of(wid * rows_per_tile * 128 + j * bsz, bsz)
                pltpu.sync_copy(dst_hbm.at[pl.ds(off, bsz)], dst_v)
                pltpu.sync_copy(rows_v, agg_sh.at[dst_v], add=True)
                return carry
            lax.fori_loop(0, n_steps, step2, 0)
            plsc.subcore_barrier()

            pltpu.sync_copy(agg_sh.at[pl.ds(base, zrows)],
                            cnt_out.at[cid, pl.ds(base, zrows)])

    k = pl.kernel(body, out_type=out_type, mesh=mesh, scratch_types=scratch)
    return k(src_flat, dst_flat, table)


def _dgT(x, w):
    return lax.dot_general(x, w, (((1,), (1,)), ((), ())),
                           preferred_element_type=jnp.float32)


def _mm_pair(x, wa, wb, b, blk=2000):
    n, d = x.shape
    h = wa.shape[0]

    def body(x_ref, wa_ref, wb_ref, b_ref, o1_ref, o2_ref):
        xb = x_ref[...]
        o1_ref[...] = _dgT(xb, wa_ref[...])
        o2_ref[...] = _dgT(xb, wb_ref[...]) + b_ref[...]

    return pl.pallas_call(
        body,
        grid=(n // blk,),
        in_specs=[pl.BlockSpec((blk, d), lambda i: (i, 0)),
                  pl.BlockSpec((h, d), lambda i: (0, 0)),
                  pl.BlockSpec((h, d), lambda i: (0, 0)),
                  pl.BlockSpec((1, h), lambda i: (0, 0))],
        out_specs=[pl.BlockSpec((blk, h), lambda i: (i, 0)),
                   pl.BlockSpec((blk, h), lambda i: (i, 0))],
        out_shape=[jax.ShapeDtypeStruct((n, h), jnp.float32),
                   jax.ShapeDtypeStruct((n, h), jnp.float32)],
    )(x, wa, wb, b.reshape(1, h))


def _combine(aggp, cnt, xr, ws, b, blk=2000):
    n, h = xr.shape

    def body(agg_ref, cnt_ref, xr_ref, *rest):
        w_refs = rest[:len(ws)]
        b_ref = rest[len(ws)]
        o_refs = rest[len(ws) + 1:]
        feats = agg_ref[0] + agg_ref[1]
        c = cnt_ref[0, :, 0:1] + cnt_ref[1, :, 0:1]
        mean = feats / jnp.maximum(c, 1.0)
        hb = jnp.maximum(mean + xr_ref[...], 0.0)
        for i, (w_ref, o_ref) in enumerate(zip(w_refs, o_refs)):
            r = _dgT(hb, w_ref[...])
            if i == len(ws) - 1:
                r = r + b_ref[...]
            o_ref[...] = r

    in_specs = [pl.BlockSpec((NC, blk, h), lambda i: (0, i, 0)),
                pl.BlockSpec((NC, blk, h), lambda i: (0, i, 0)),
                pl.BlockSpec((blk, h), lambda i: (i, 0))]
    in_specs += [pl.BlockSpec((w.shape[0], h), lambda i: (0, 0)) for w in ws]
    in_specs.append(pl.BlockSpec((1, ws[-1].shape[0]), lambda i: (0, 0)))

    return pl.pallas_call(
        body,
        grid=(n // blk,),
        in_specs=in_specs,
        out_specs=[pl.BlockSpec((blk, w.shape[0]), lambda i: (i, 0)) for w in ws],
        out_shape=[jax.ShapeDtypeStruct((n, w.shape[0]), jnp.float32) for w in ws],
    )(aggp, cnt, xr, *ws, b.reshape(1, -1))


def kernel(x, edge_index, W1l, W1r, b1, W2l, W2r, b2, Wc, bc):
    n, d = x.shape
    e = edge_index.shape[1]

    rows_per_tile = -(-(-(-e // (NW * 128))) // 16) * 16
    e_pad = rows_per_tile * NW * 128
    n_pad = -(-(n + 1) // (NS * 8)) * (NS * 8)

    src = edge_index[0]
    dst = edge_index[1]
    pad = e_pad - e
    src_flat = jnp.concatenate([src, jnp.zeros((pad,), jnp.int32)])
    dst_flat = jnp.concatenate([dst, jnp.full((pad,), n, jnp.int32)])

    xl1, xr1 = _mm_pair(x, W1l, W1r, b1)
    agg1, cnt = _sc_segsum(src_flat, dst_flat, xl1, n_pad, rows_per_tile, True)
    xl2, xr2 = _combine(agg1, cnt, xr1, [W2l, W2r], b2)
    agg2 = _sc_segsum(src_flat, dst_flat, xl2, n_pad, rows_per_tile, False)[0]
    out = _combine(agg2, cnt, xr2, [Wc], bc)[0]
    return out
